# baseline (speedup 1.0000x reference)
import jax
import jax.numpy as jnp
from jax import lax
from jax.experimental import pallas as pl
from jax.experimental.pallas import tpu as pltpu
from jax.experimental.pallas import tpu_sc as plsc

N = 10000
E = 160000
DIN = 256
DOUT = 256
Q = 64
NT = 16
L = 16
EPT = E // NT
G = 128
CH = 80
EPTP = CH * G
NP = 10240
RPT = NP // NT
NB = RPT // G
NSTEP = CH // 2
BN = 1000


def _tc_body(x_ref, wt_ref, a1_ref, a2_ref,
             z0_ref, z1_ref, z2_ref, z3_ref, s_ref, d_ref):
    z = jnp.dot(x_ref[...], wt_ref[...], preferred_element_type=jnp.float32)
    z0_ref[...] = z[:, 0 * Q:1 * Q]
    z1_ref[...] = z[:, 1 * Q:2 * Q]
    z2_ref[...] = z[:, 2 * Q:3 * Q]
    z3_ref[...] = z[:, 3 * Q:4 * Q]
    s_ref[...] = jnp.dot(z, a1_ref[...], preferred_element_type=jnp.float32)
    d_ref[...] = jnp.dot(z, a2_ref[...], preferred_element_type=jnp.float32)


def _sc_body(z0_h, z1_h, z2_h, z3_h, s_h, d_h, src_h, dst_h,
             o0_h, o1_h, o2_h, o3_h,
             s_v, d_v, src_v, dst_v, ee_v, invsl_v, alpha_v,
             gbuf0, gbuf1, hacc_s, den_s,
             semd, semg0, semg1, sems0, sems1):
    cid = lax.axis_index("c")
    sid = lax.axis_index("s")

    pltpu.sync_copy(s_h, s_v)
    pltpu.sync_copy(d_h, d_v)
    pltpu.sync_copy(src_h.at[sid], src_v)
    pltpu.sync_copy(dst_h.at[sid], dst_v)

    zero16 = jnp.zeros((L,), jnp.float32)
    iota16 = lax.iota(jnp.int32, L)

    def zero_invsl(i, c):
        invsl_v[pl.ds(i * L, L)] = zero16
        return c
    lax.fori_loop(0, RPT // L, zero_invsl, 0)
    pltpu.sync_copy(invsl_v, den_s.at[pl.ds(sid * RPT, RPT)])

    neg = jnp.full((L,), -1e30, jnp.float32)

    def mxs(i, acc):
        return jnp.maximum(acc, s_v[pl.ds(i * L, L)])

    def mxd(i, acc):
        return jnp.maximum(acc, d_v[pl.ds(i * L, L)])

    def lane_max(v):
        m = v[0]
        for i in range(1, L):
            m = jnp.maximum(m, v[i])
        return m
    M = lane_max(lax.fori_loop(0, NP // L, mxs, neg)) + \
        lane_max(lax.fori_loop(0, NP // L, mxd, neg))

    plsc.subcore_barrier()

    def ph1(j, c):
        for k in range(G // L):
            sl = pl.ds(k * L, L)
            s16 = src_v[j, sl]
            d16 = dst_v[j, sl]
            t = plsc.load_gather(s_v, [s16]) + plsc.load_gather(d_v, [d16])
            e = jnp.where(t >= 0, t, 0.01 * t)
            ee = jnp.exp(e - M)
            lidx = j * G + k * L + iota16
            ee = jnp.where(lidx < EPT, ee, 0.0)
            ee_v[j, sl] = ee

        @pl.when(j > 0)
        def _():
            pltpu.make_async_copy(ee_v.at[j], den_s.at[dst_v.at[j]],
                                  semd).wait()
        pltpu.async_copy(ee_v.at[j], den_s.at[dst_v.at[j]], semd, add=True)
        return c
    lax.fori_loop(0, CH, ph1, 0)
    pltpu.make_async_copy(ee_v.at[0], den_s.at[dst_v.at[0]], semd).wait()
    plsc.subcore_barrier()

    pltpu.sync_copy(den_s.at[pl.ds(sid * RPT, RPT)], invsl_v)

    def inv_loop(v, c):
        sl = pl.ds(v * L, L)
        acc = invsl_v[sl]
        invsl_v[sl] = jnp.where(acc > 0, 1.0 / acc, 1.0)
        return c
    lax.fori_loop(0, RPT // L, inv_loop, 0)
    pltpu.sync_copy(invsl_v, den_s.at[pl.ds(sid * RPT, RPT)])
    plsc.subcore_barrier()
    pltpu.sync_copy(den_s, s_v)

    def mk_alpha(j):
        for k in range(G // L):
            sl = pl.ds(k * L, L)
            iv = plsc.load_gather(s_v, [dst_v[j, sl]])
            alpha_v[sl] = ee_v[j, sl] * iv

    def scale(buf):
        def sc_g(g, cc):
            a16 = alpha_v[pl.ds(g * L, L)]
            for r in range(L):
                ab = jnp.full((L,), a16[r])
                row = g * L + r
                for v in range(Q // L):
                    sl2 = pl.ds(v * L, L)
                    buf[row, sl2] = buf[row, sl2] * ab
            return cc
        lax.fori_loop(0, G // L, sc_g, 0)

    def phase2(z_h, out_h):
        def zg(r, c):
            for v in range(Q // L):
                gbuf0[r, pl.ds(v * L, L)] = zero16
            return c
        lax.fori_loop(0, G, zg, 0)
        for b in range(NB):
            pltpu.sync_copy(gbuf0, hacc_s.at[pl.ds(sid * RPT + b * G, G)])
        plsc.subcore_barrier()

        pltpu.async_copy(z_h.at[src_v.at[0]], gbuf0, semg0)

        def step(i, c):
            j0 = 2 * i
            j1 = 2 * i + 1
            mk_alpha(j0)
            pltpu.make_async_copy(z_h.at[src_v.at[j0]], gbuf0, semg0).wait()
            scale(gbuf0)

            @pl.when(i > 0)
            def _():
                pltpu.make_async_copy(gbuf1, hacc_s.at[dst_v.at[j1]],
                                      sems1).wait()
            pltpu.async_copy(z_h.at[src_v.at[j1]], gbuf1, semg1)
            pltpu.async_copy(gbuf0, hacc_s.at[dst_v.at[j0]], sems0, add=True)
            mk_alpha(j1)
            pltpu.make_async_copy(z_h.at[src_v.at[j1]], gbuf1, semg1).wait()
            scale(gbuf1)
            pltpu.make_async_copy(gbuf0, hacc_s.at[dst_v.at[j0]],
                                  sems0).wait()

            @pl.when(i < NSTEP - 1)
            def _():
                pltpu.async_copy(z_h.at[src_v.at[j0 + 2]], gbuf0, semg0)
            pltpu.async_copy(gbuf1, hacc_s.at[dst_v.at[j1]], sems1, add=True)
            return c
        lax.fori_loop(0, NSTEP, step, 0)
        pltpu.make_async_copy(gbuf1, hacc_s.at[dst_v.at[CH - 1]],
                              sems1).wait()
        plsc.subcore_barrier()
        pltpu.sync_copy(hacc_s.at[pl.ds(sid * RPT, RPT)],
                        out_h.at[pl.ds(sid * RPT, RPT)])

    @pl.when(cid == 0)
    def _():
        phase2(z0_h, o0_h)
        phase2(z1_h, o1_h)

    @pl.when(cid == 1)
    def _():
        phase2(z2_h, o2_h)
        phase2(z3_h, o3_h)


def kernel(x, edge_index, W, A):
    Wt = W.T
    a1 = A[0, :DOUT].reshape(DOUT, 1)
    a2 = A[0, DOUT:].reshape(DOUT, 1)
    zq = pl.pallas_call(
        _tc_body,
        grid=(N // BN,),
        in_specs=[pl.BlockSpec((BN, DIN), lambda i: (i, 0)),
                  pl.BlockSpec((DIN, DOUT), lambda i: (0, 0)),
                  pl.BlockSpec((DOUT, 1), lambda i: (0, 0)),
                  pl.BlockSpec((DOUT, 1), lambda i: (0, 0))],
        out_specs=[pl.BlockSpec((BN, Q), lambda i: (i, 0))] * 4 +
                  [pl.BlockSpec((BN, 1), lambda i: (i, 0))] * 2,
        out_shape=[jax.ShapeDtypeStruct((N, Q), jnp.float32)] * 4 +
                  [jax.ShapeDtypeStruct((N, 1), jnp.float32)] * 2,
    )(x, Wt, a1, a2)
    z0, z1, z2, z3, s2, d2 = zq

    s = jnp.pad(s2[:, 0], (0, NP - N))
    d = jnp.pad(d2[:, 0], (0, NP - N))
    src = jnp.pad(edge_index[0].reshape(NT, EPT),
                  ((0, 0), (0, EPTP - EPT))).reshape(NT, CH, G)
    dst = jnp.pad(edge_index[1].reshape(NT, EPT),
                  ((0, 0), (0, EPTP - EPT))).reshape(NT, CH, G)

    sc = pl.kernel(
        _sc_body,
        out_type=[jax.ShapeDtypeStruct((NP, Q), jnp.float32)] * 4,
        mesh=plsc.VectorSubcoreMesh(core_axis_name="c", subcore_axis_name="s"),
        compiler_params=pltpu.CompilerParams(needs_layout_passes=False,
                                             use_tc_tiling_on_sc=False),
        scratch_types=[
            pltpu.VMEM((NP,), jnp.float32),
            pltpu.VMEM((NP,), jnp.float32),
            pltpu.VMEM((CH, G), jnp.int32),
            pltpu.VMEM((CH, G), jnp.int32),
            pltpu.VMEM((CH, G), jnp.float32),
            pltpu.VMEM((RPT,), jnp.float32),
            pltpu.VMEM((G,), jnp.float32),
            pltpu.VMEM((G, Q), jnp.float32),
            pltpu.VMEM((G, Q), jnp.float32),
            pltpu.VMEM_SHARED((NP, Q), jnp.float32),
            pltpu.VMEM_SHARED((NP,), jnp.float32),
            pltpu.SemaphoreType.DMA,
            pltpu.SemaphoreType.DMA,
            pltpu.SemaphoreType.DMA,
            pltpu.SemaphoreType.DMA,
            pltpu.SemaphoreType.DMA,
        ],
    )
    o0, o1, o2, o3 = sc(z0, z1, z2, z3, s, d, src, dst)
    return jnp.concatenate([o0[:N], o1[:N], o2[:N], o3[:N]], axis=1)


# speedup vs baseline: 1.0004x; 1.0004x over previous
import jax
import jax.numpy as jnp
from jax import lax
from jax.experimental import pallas as pl
from jax.experimental.pallas import tpu as pltpu
from jax.experimental.pallas import tpu_sc as plsc

N = 10000
E = 160000
DIN = 256
DOUT = 256
Q = 64
NT = 16
L = 16
EPT = E // NT
G = 128
CH = 80
EPTP = CH * G
NP = 10240
RPT = NP // NT
NB = RPT // G
NSTEP = CH // 2
BN = 1000


def _tc_body(x_ref, wt_ref, a1_ref, a2_ref,
             z0_ref, z1_ref, z2_ref, z3_ref, s_ref, d_ref):
    z = jnp.dot(x_ref[...], wt_ref[...], preferred_element_type=jnp.float32)
    z0_ref[...] = z[:, 0 * Q:1 * Q]
    z1_ref[...] = z[:, 1 * Q:2 * Q]
    z2_ref[...] = z[:, 2 * Q:3 * Q]
    z3_ref[...] = z[:, 3 * Q:4 * Q]
    s_ref[...] = jnp.dot(z, a1_ref[...], preferred_element_type=jnp.float32)
    d_ref[...] = jnp.dot(z, a2_ref[...], preferred_element_type=jnp.float32)


def _sc_body(z0_h, z1_h, z2_h, z3_h, s_h, d_h, src_h, dst_h,
             o0_h, o1_h, o2_h, o3_h,
             s_v, d_v, src_v, dst_v, ee_v, invsl_v, alpha_v,
             gbuf0, gbuf1, hacc_s, den_s,
             semd, semg0, semg1, sems0, sems1):
    cid = lax.axis_index("c")
    sid = lax.axis_index("s")

    pltpu.sync_copy(s_h, s_v)
    pltpu.sync_copy(d_h, d_v)
    pltpu.sync_copy(src_h.at[sid], src_v)
    pltpu.sync_copy(dst_h.at[sid], dst_v)

    zero16 = jnp.zeros((L,), jnp.float32)
    iota16 = lax.iota(jnp.int32, L)

    def zero_invsl(i, c):
        invsl_v[pl.ds(i * L, L)] = zero16
        return c
    lax.fori_loop(0, RPT // L, zero_invsl, 0)
    pltpu.sync_copy(invsl_v, den_s.at[pl.ds(sid * RPT, RPT)])

    neg = jnp.full((L,), -1e30, jnp.float32)

    def mxs(i, acc):
        return jnp.maximum(acc, s_v[pl.ds(i * L, L)])

    def mxd(i, acc):
        return jnp.maximum(acc, d_v[pl.ds(i * L, L)])

    def lane_max(v):
        m = v[0]
        for i in range(1, L):
            m = jnp.maximum(m, v[i])
        return m
    M = lane_max(lax.fori_loop(0, NP // L, mxs, neg)) + \
        lane_max(lax.fori_loop(0, NP // L, mxd, neg))

    plsc.subcore_barrier()

    def ph1(j, c):
        for k in range(G // L):
            sl = pl.ds(k * L, L)
            s16 = src_v[j, sl]
            d16 = dst_v[j, sl]
            t = plsc.load_gather(s_v, [s16]) + plsc.load_gather(d_v, [d16])
            e = jnp.where(t >= 0, t, 0.01 * t)
            ee = jnp.exp(e - M)
            lidx = j * G + k * L + iota16
            ee = jnp.where(lidx < EPT, ee, 0.0)
            ee_v[j, sl] = ee

        @pl.when(j > 0)
        def _():
            pltpu.make_async_copy(ee_v.at[j], den_s.at[dst_v.at[j]],
                                  semd).wait()
        pltpu.async_copy(ee_v.at[j], den_s.at[dst_v.at[j]], semd, add=True)
        return c
    lax.fori_loop(0, CH, ph1, 0)
    pltpu.make_async_copy(ee_v.at[0], den_s.at[dst_v.at[0]], semd).wait()
    plsc.subcore_barrier()

    pltpu.sync_copy(den_s.at[pl.ds(sid * RPT, RPT)], invsl_v)

    def inv_loop(v, c):
        sl = pl.ds(v * L, L)
        acc = invsl_v[sl]
        invsl_v[sl] = jnp.where(acc > 0, 1.0 / acc, 1.0)
        return c
    lax.fori_loop(0, RPT // L, inv_loop, 0)
    pltpu.sync_copy(invsl_v, den_s.at[pl.ds(sid * RPT, RPT)])
    plsc.subcore_barrier()
    pltpu.sync_copy(den_s, s_v)

    def mk_alpha(j):
        for k in range(G // L):
            sl = pl.ds(k * L, L)
            iv = plsc.load_gather(s_v, [dst_v[j, sl]])
            alpha_v[sl] = ee_v[j, sl] * iv

    gdn = lax.GatherDimensionNumbers(offset_dims=(), collapsed_slice_dims=(0,),
                                     start_index_map=(0,))

    def scale(buf):
        def sc_g(g, cc):
            a16 = alpha_v[pl.ds(g * L, L)]
            for r in range(L):
                ab = lax.gather(a16, jnp.full((L, 1), r, jnp.int32), gdn,
                                (1,),
                                mode=lax.GatherScatterMode.PROMISE_IN_BOUNDS)
                row = g * L + r
                for v in range(Q // L):
                    sl2 = pl.ds(v * L, L)
                    buf[row, sl2] = buf[row, sl2] * ab
            return cc
        lax.fori_loop(0, G // L, sc_g, 0)

    def phase2(z_h, out_h):
        def zg(r, c):
            for v in range(Q // L):
                gbuf0[r, pl.ds(v * L, L)] = zero16
            return c
        lax.fori_loop(0, G, zg, 0)
        for b in range(NB):
            pltpu.sync_copy(gbuf0, hacc_s.at[pl.ds(sid * RPT + b * G, G)])
        plsc.subcore_barrier()

        pltpu.async_copy(z_h.at[src_v.at[0]], gbuf0, semg0)

        def step(i, c):
            j0 = 2 * i
            j1 = 2 * i + 1
            mk_alpha(j0)
            pltpu.make_async_copy(z_h.at[src_v.at[j0]], gbuf0, semg0).wait()
            scale(gbuf0)

            @pl.when(i > 0)
            def _():
                pltpu.make_async_copy(gbuf1, hacc_s.at[dst_v.at[j1]],
                                      sems1).wait()
            pltpu.async_copy(z_h.at[src_v.at[j1]], gbuf1, semg1)
            pltpu.async_copy(gbuf0, hacc_s.at[dst_v.at[j0]], sems0, add=True)
            mk_alpha(j1)
            pltpu.make_async_copy(z_h.at[src_v.at[j1]], gbuf1, semg1).wait()
            scale(gbuf1)
            pltpu.make_async_copy(gbuf0, hacc_s.at[dst_v.at[j0]],
                                  sems0).wait()

            @pl.when(i < NSTEP - 1)
            def _():
                pltpu.async_copy(z_h.at[src_v.at[j0 + 2]], gbuf0, semg0)
            pltpu.async_copy(gbuf1, hacc_s.at[dst_v.at[j1]], sems1, add=True)
            return c
        lax.fori_loop(0, NSTEP, step, 0)
        pltpu.make_async_copy(gbuf1, hacc_s.at[dst_v.at[CH - 1]],
                              sems1).wait()
        plsc.subcore_barrier()
        pltpu.sync_copy(hacc_s.at[pl.ds(sid * RPT, RPT)],
                        out_h.at[pl.ds(sid * RPT, RPT)])

    @pl.when(cid == 0)
    def _():
        phase2(z0_h, o0_h)
        phase2(z1_h, o1_h)

    @pl.when(cid == 1)
    def _():
        phase2(z2_h, o2_h)
        phase2(z3_h, o3_h)


def kernel(x, edge_index, W, A):
    Wt = W.T
    a1 = A[0, :DOUT].reshape(DOUT, 1)
    a2 = A[0, DOUT:].reshape(DOUT, 1)
    zq = pl.pallas_call(
        _tc_body,
        grid=(N // BN,),
        in_specs=[pl.BlockSpec((BN, DIN), lambda i: (i, 0)),
                  pl.BlockSpec((DIN, DOUT), lambda i: (0, 0)),
                  pl.BlockSpec((DOUT, 1), lambda i: (0, 0)),
                  pl.BlockSpec((DOUT, 1), lambda i: (0, 0))],
        out_specs=[pl.BlockSpec((BN, Q), lambda i: (i, 0))] * 4 +
                  [pl.BlockSpec((BN, 1), lambda i: (i, 0))] * 2,
        out_shape=[jax.ShapeDtypeStruct((N, Q), jnp.float32)] * 4 +
                  [jax.ShapeDtypeStruct((N, 1), jnp.float32)] * 2,
    )(x, Wt, a1, a2)
    z0, z1, z2, z3, s2, d2 = zq

    s = jnp.pad(s2[:, 0], (0, NP - N))
    d = jnp.pad(d2[:, 0], (0, NP - N))
    src = jnp.pad(edge_index[0].reshape(NT, EPT),
                  ((0, 0), (0, EPTP - EPT))).reshape(NT, CH, G)
    dst = jnp.pad(edge_index[1].reshape(NT, EPT),
                  ((0, 0), (0, EPTP - EPT))).reshape(NT, CH, G)

    sc = pl.kernel(
        _sc_body,
        out_type=[jax.ShapeDtypeStruct((NP, Q), jnp.float32)] * 4,
        mesh=plsc.VectorSubcoreMesh(core_axis_name="c", subcore_axis_name="s"),
        compiler_params=pltpu.CompilerParams(needs_layout_passes=False,
                                             use_tc_tiling_on_sc=False),
        scratch_types=[
            pltpu.VMEM((NP,), jnp.float32),
            pltpu.VMEM((NP,), jnp.float32),
            pltpu.VMEM((CH, G), jnp.int32),
            pltpu.VMEM((CH, G), jnp.int32),
            pltpu.VMEM((CH, G), jnp.float32),
            pltpu.VMEM((RPT,), jnp.float32),
            pltpu.VMEM((G,), jnp.float32),
            pltpu.VMEM((G, Q), jnp.float32),
            pltpu.VMEM((G, Q), jnp.float32),
            pltpu.VMEM_SHARED((NP, Q), jnp.float32),
            pltpu.VMEM_SHARED((NP,), jnp.float32),
            pltpu.SemaphoreType.DMA,
            pltpu.SemaphoreType.DMA,
            pltpu.SemaphoreType.DMA,
            pltpu.SemaphoreType.DMA,
            pltpu.SemaphoreType.DMA,
        ],
    )
    o0, o1, o2, o3 = sc(z0, z1, z2, z3, s, d, src, dst)
    return jnp.concatenate([o0[:N], o1[:N], o2[:N], o3[:N]], axis=1)


# speedup vs baseline: 1.5440x; 1.5434x over previous
import jax
import jax.numpy as jnp
from jax import lax
from jax.experimental import pallas as pl
from jax.experimental.pallas import tpu as pltpu
from jax.experimental.pallas import tpu_sc as plsc

N = 10000
E = 160000
DIN = 256
DOUT = 256
Q = 64
NT = 16
L = 16
EPT = E // NT
G = 128
CH = 80
EPTP = CH * G
NP = 10240
RPT = NP // NT
NB = RPT // G
NSTEP = CH // 2
BN = 1000


def _tc_body(x_ref, wt_ref, a1_ref, a2_ref,
             z0_ref, z1_ref, z2_ref, z3_ref, s_ref, d_ref):
    z = jnp.dot(x_ref[...], wt_ref[...], preferred_element_type=jnp.float32)
    z0_ref[...] = z[:, 0 * Q:1 * Q]
    z1_ref[...] = z[:, 1 * Q:2 * Q]
    z2_ref[...] = z[:, 2 * Q:3 * Q]
    z3_ref[...] = z[:, 3 * Q:4 * Q]
    s_ref[...] = jnp.dot(z, a1_ref[...], preferred_element_type=jnp.float32)
    d_ref[...] = jnp.dot(z, a2_ref[...], preferred_element_type=jnp.float32)


def _sc_body(z0_h, z1_h, z2_h, z3_h, s_h, d_h, src_h, dst_h,
             o0_h, o1_h, o2_h, o3_h,
             s_v, d_v, src_v, dst_v, ee_v, invsl_v,
             gbuf0, gbuf1, hacc_s, den_s,
             semd, semg0, semg1, sems0, sems1):
    cid = lax.axis_index("c")
    sid = lax.axis_index("s")

    pltpu.sync_copy(s_h, s_v)
    pltpu.sync_copy(d_h, d_v)
    pltpu.sync_copy(src_h.at[sid], src_v)
    pltpu.sync_copy(dst_h.at[sid], dst_v)

    zero16 = jnp.zeros((L,), jnp.float32)
    iota16 = lax.iota(jnp.int32, L)

    def zero_invsl(i, c):
        invsl_v[pl.ds(i * L, L)] = zero16
        return c
    lax.fori_loop(0, RPT // L, zero_invsl, 0)
    pltpu.sync_copy(invsl_v, den_s.at[pl.ds(sid * RPT, RPT)])

    neg = jnp.full((L,), -1e30, jnp.float32)

    def mxs(i, acc):
        return jnp.maximum(acc, s_v[pl.ds(i * L, L)])

    def mxd(i, acc):
        return jnp.maximum(acc, d_v[pl.ds(i * L, L)])

    def lane_max(v):
        m = v[0]
        for i in range(1, L):
            m = jnp.maximum(m, v[i])
        return m
    M = lane_max(lax.fori_loop(0, NP // L, mxs, neg)) + \
        lane_max(lax.fori_loop(0, NP // L, mxd, neg))

    plsc.subcore_barrier()

    def ph1(j, c):
        for k in range(G // L):
            sl = pl.ds(k * L, L)
            s16 = src_v[j, sl]
            d16 = dst_v[j, sl]
            t = plsc.load_gather(s_v, [s16]) + plsc.load_gather(d_v, [d16])
            e = jnp.where(t >= 0, t, 0.01 * t)
            ee = jnp.exp(e - M)
            lidx = j * G + k * L + iota16
            ee = jnp.where(lidx < EPT, ee, 0.0)
            ee_v[j, sl] = ee

        @pl.when(j > 0)
        def _():
            pltpu.make_async_copy(ee_v.at[j], den_s.at[dst_v.at[j]],
                                  semd).wait()
        pltpu.async_copy(ee_v.at[j], den_s.at[dst_v.at[j]], semd, add=True)
        return c
    lax.fori_loop(0, CH, ph1, 0)
    pltpu.make_async_copy(ee_v.at[0], den_s.at[dst_v.at[0]], semd).wait()
    plsc.subcore_barrier()

    pltpu.sync_copy(den_s.at[pl.ds(sid * RPT, RPT)], invsl_v)

    def inv_loop(v, c):
        sl = pl.ds(v * L, L)
        acc = invsl_v[sl]
        invsl_v[sl] = jnp.where(acc > 0, 1.0 / acc, 1.0)
        return c
    lax.fori_loop(0, RPT // L, inv_loop, 0)
    pltpu.sync_copy(invsl_v, den_s.at[pl.ds(sid * RPT, RPT)])
    plsc.subcore_barrier()
    pltpu.sync_copy(den_s, s_v)

    @plsc.parallel_loop(0, CH, unroll=2)
    def _(j):
        for k in range(G // L):
            sl = pl.ds(k * L, L)
            iv = plsc.load_gather(s_v, [dst_v[j, sl]])
            ee_v[j, sl] = ee_v[j, sl] * iv

    gdn = lax.GatherDimensionNumbers(offset_dims=(), collapsed_slice_dims=(0,),
                                     start_index_map=(0,))

    def scale(buf, j):
        @plsc.parallel_loop(0, G // L, unroll=2)
        def _(g):
            a16 = ee_v[j, pl.ds(g * L, L)]
            for r in range(L):
                ab = lax.gather(a16, jnp.full((L, 1), r, jnp.int32), gdn,
                                (1,),
                                mode=lax.GatherScatterMode.PROMISE_IN_BOUNDS)
                row = g * L + r
                for v in range(Q // L):
                    sl2 = pl.ds(v * L, L)
                    buf[row, sl2] = buf[row, sl2] * ab

    def phase2(z_h, out_h):
        def zg(r, c):
            for v in range(Q // L):
                gbuf0[r, pl.ds(v * L, L)] = zero16
            return c
        lax.fori_loop(0, G, zg, 0)
        for b in range(NB):
            pltpu.sync_copy(gbuf0, hacc_s.at[pl.ds(sid * RPT + b * G, G)])
        plsc.subcore_barrier()

        pltpu.async_copy(z_h.at[src_v.at[0]], gbuf0, semg0)

        def step(i, c):
            j0 = 2 * i
            j1 = 2 * i + 1
            pltpu.make_async_copy(z_h.at[src_v.at[j0]], gbuf0, semg0).wait()
            scale(gbuf0, j0)

            @pl.when(i > 0)
            def _():
                pltpu.make_async_copy(gbuf1, hacc_s.at[dst_v.at[j1]],
                                      sems1).wait()
            pltpu.async_copy(z_h.at[src_v.at[j1]], gbuf1, semg1)
            pltpu.async_copy(gbuf0, hacc_s.at[dst_v.at[j0]], sems0, add=True)
            pltpu.make_async_copy(z_h.at[src_v.at[j1]], gbuf1, semg1).wait()
            scale(gbuf1, j1)
            pltpu.make_async_copy(gbuf0, hacc_s.at[dst_v.at[j0]],
                                  sems0).wait()

            @pl.when(i < NSTEP - 1)
            def _():
                pltpu.async_copy(z_h.at[src_v.at[j0 + 2]], gbuf0, semg0)
            pltpu.async_copy(gbuf1, hacc_s.at[dst_v.at[j1]], sems1, add=True)
            return c
        lax.fori_loop(0, NSTEP, step, 0)
        pltpu.make_async_copy(gbuf1, hacc_s.at[dst_v.at[CH - 1]],
                              sems1).wait()
        plsc.subcore_barrier()
        pltpu.sync_copy(hacc_s.at[pl.ds(sid * RPT, RPT)],
                        out_h.at[pl.ds(sid * RPT, RPT)])

    @pl.when(cid == 0)
    def _():
        phase2(z0_h, o0_h)
        phase2(z1_h, o1_h)

    @pl.when(cid == 1)
    def _():
        phase2(z2_h, o2_h)
        phase2(z3_h, o3_h)


def kernel(x, edge_index, W, A):
    Wt = W.T
    a1 = A[0, :DOUT].reshape(DOUT, 1)
    a2 = A[0, DOUT:].reshape(DOUT, 1)
    zq = pl.pallas_call(
        _tc_body,
        grid=(N // BN,),
        in_specs=[pl.BlockSpec((BN, DIN), lambda i: (i, 0)),
                  pl.BlockSpec((DIN, DOUT), lambda i: (0, 0)),
                  pl.BlockSpec((DOUT, 1), lambda i: (0, 0)),
                  pl.BlockSpec((DOUT, 1), lambda i: (0, 0))],
        out_specs=[pl.BlockSpec((BN, Q), lambda i: (i, 0))] * 4 +
                  [pl.BlockSpec((BN, 1), lambda i: (i, 0))] * 2,
        out_shape=[jax.ShapeDtypeStruct((N, Q), jnp.float32)] * 4 +
                  [jax.ShapeDtypeStruct((N, 1), jnp.float32)] * 2,
    )(x, Wt, a1, a2)
    z0, z1, z2, z3, s2, d2 = zq

    s = jnp.pad(s2[:, 0], (0, NP - N))
    d = jnp.pad(d2[:, 0], (0, NP - N))
    src = jnp.pad(edge_index[0].reshape(NT, EPT),
                  ((0, 0), (0, EPTP - EPT))).reshape(NT, CH, G)
    dst = jnp.pad(edge_index[1].reshape(NT, EPT),
                  ((0, 0), (0, EPTP - EPT))).reshape(NT, CH, G)

    sc = pl.kernel(
        _sc_body,
        out_type=[jax.ShapeDtypeStruct((NP, Q), jnp.float32)] * 4,
        mesh=plsc.VectorSubcoreMesh(core_axis_name="c", subcore_axis_name="s"),
        compiler_params=pltpu.CompilerParams(needs_layout_passes=False,
                                             use_tc_tiling_on_sc=False),
        scratch_types=[
            pltpu.VMEM((NP,), jnp.float32),
            pltpu.VMEM((NP,), jnp.float32),
            pltpu.VMEM((CH, G), jnp.int32),
            pltpu.VMEM((CH, G), jnp.int32),
            pltpu.VMEM((CH, G), jnp.float32),
            pltpu.VMEM((RPT,), jnp.float32),
            pltpu.VMEM((G, Q), jnp.float32),
            pltpu.VMEM((G, Q), jnp.float32),
            pltpu.VMEM_SHARED((NP, Q), jnp.float32),
            pltpu.VMEM_SHARED((NP,), jnp.float32),
            pltpu.SemaphoreType.DMA,
            pltpu.SemaphoreType.DMA,
            pltpu.SemaphoreType.DMA,
            pltpu.SemaphoreType.DMA,
            pltpu.SemaphoreType.DMA,
        ],
    )
    o0, o1, o2, o3 = sc(z0, z1, z2, z3, s, d, src, dst)
    return jnp.concatenate([o0[:N], o1[:N], o2[:N], o3[:N]], axis=1)


# speedup vs baseline: 1.9128x; 1.2389x over previous
import jax
import jax.numpy as jnp
from jax import lax
from jax.experimental import pallas as pl
from jax.experimental.pallas import tpu as pltpu
from jax.experimental.pallas import tpu_sc as plsc

N = 10000
E = 160000
DIN = 256
DOUT = 256
Q = 32
NZ = DOUT // Q
NT = 16
L = 16
EPT = E // NT
G = 128
CH = 80
EPTP = CH * G
NP = 10240
RPT = NP // NT
NRS = N // NT
NB = RPT // G
NSTEP = CH // 2
BN = 1000


def _tc_body(x_ref, wt_ref, a1_ref, a2_ref, *out_refs):
    z = jnp.dot(x_ref[...], wt_ref[...], preferred_element_type=jnp.float32)
    for q in range(NZ):
        out_refs[q][...] = z[:, q * Q:(q + 1) * Q]
    out_refs[NZ][...] = jnp.dot(z, a1_ref[...],
                                preferred_element_type=jnp.float32)
    out_refs[NZ + 1][...] = jnp.dot(z, a2_ref[...],
                                    preferred_element_type=jnp.float32)


def _sc_body(*refs):
    (z0_h, z1_h, z2_h, z3_h, z4_h, z5_h, z6_h, z7_h, s_h, d_h, src_h, dst_h,
     o0_h, o1_h, o2_h, o3_h, o4_h, o5_h, o6_h, o7_h,
     s_v, d_v, src_v, dst_v, ee_v, invsl_v, gbuf0, gbuf1,
     zst_s, hacc_s, den_s, semd, semg0, semg1, sems0, sems1) = refs
    cid = lax.axis_index("c")
    sid = lax.axis_index("s")

    pltpu.sync_copy(s_h, s_v)
    pltpu.sync_copy(d_h, d_v)
    pltpu.sync_copy(src_h.at[sid], src_v)
    pltpu.sync_copy(dst_h.at[sid], dst_v)

    zero16 = jnp.zeros((L,), jnp.float32)
    iota16 = lax.iota(jnp.int32, L)

    def zero_invsl(i, c):
        invsl_v[pl.ds(i * L, L)] = zero16
        return c
    lax.fori_loop(0, RPT // L, zero_invsl, 0)
    pltpu.sync_copy(invsl_v, den_s.at[pl.ds(sid * RPT, RPT)])

    neg = jnp.full((L,), -1e30, jnp.float32)

    def mxs(i, acc):
        return jnp.maximum(acc, s_v[pl.ds(i * L, L)])

    def mxd(i, acc):
        return jnp.maximum(acc, d_v[pl.ds(i * L, L)])

    def lane_max(v):
        m = v[0]
        for i in range(1, L):
            m = jnp.maximum(m, v[i])
        return m
    M = lane_max(lax.fori_loop(0, NP // L, mxs, neg)) + \
        lane_max(lax.fori_loop(0, NP // L, mxd, neg))

    plsc.subcore_barrier()

    def ph1(j, c):
        for k in range(G // L):
            sl = pl.ds(k * L, L)
            s16 = src_v[j, sl]
            d16 = dst_v[j, sl]
            t = plsc.load_gather(s_v, [s16]) + plsc.load_gather(d_v, [d16])
            e = jnp.where(t >= 0, t, 0.01 * t)
            ee = jnp.exp(e - M)
            lidx = j * G + k * L + iota16
            ee = jnp.where(lidx < EPT, ee, 0.0)
            ee_v[j, sl] = ee

        @pl.when(j > 0)
        def _():
            pltpu.make_async_copy(ee_v.at[j], den_s.at[dst_v.at[j]],
                                  semd).wait()
        pltpu.async_copy(ee_v.at[j], den_s.at[dst_v.at[j]], semd, add=True)
        return c
    lax.fori_loop(0, CH, ph1, 0)
    pltpu.make_async_copy(ee_v.at[0], den_s.at[dst_v.at[0]], semd).wait()
    plsc.subcore_barrier()

    pltpu.sync_copy(den_s.at[pl.ds(sid * RPT, RPT)], invsl_v)

    def inv_loop(v, c):
        sl = pl.ds(v * L, L)
        acc = invsl_v[sl]
        invsl_v[sl] = jnp.where(acc > 0, 1.0 / acc, 1.0)
        return c
    lax.fori_loop(0, RPT // L, inv_loop, 0)
    pltpu.sync_copy(invsl_v, den_s.at[pl.ds(sid * RPT, RPT)])
    plsc.subcore_barrier()
    pltpu.sync_copy(den_s, s_v)

    @plsc.parallel_loop(0, CH, unroll=2)
    def _(j):
        for k in range(G // L):
            sl = pl.ds(k * L, L)
            iv = plsc.load_gather(s_v, [dst_v[j, sl]])
            ee_v[j, sl] = ee_v[j, sl] * iv

    gdn = lax.GatherDimensionNumbers(offset_dims=(), collapsed_slice_dims=(0,),
                                     start_index_map=(0,))

    def scale(buf, j):
        @plsc.parallel_loop(0, G // L, unroll=2)
        def _(g):
            a16 = ee_v[j, pl.ds(g * L, L)]
            for r in range(L):
                ab = lax.gather(a16, jnp.full((L, 1), r, jnp.int32), gdn,
                                (1,),
                                mode=lax.GatherScatterMode.PROMISE_IN_BOUNDS)
                row = g * L + r
                for v in range(Q // L):
                    sl2 = pl.ds(v * L, L)
                    buf[row, sl2] = buf[row, sl2] * ab

    def phase2(z_h, out_h):
        def zg(r, c):
            for v in range(Q // L):
                gbuf0[r, pl.ds(v * L, L)] = zero16
            return c
        lax.fori_loop(0, G, zg, 0)
        for b in range(NB):
            pltpu.sync_copy(gbuf0, hacc_s.at[pl.ds(sid * RPT + b * G, G)])
        pltpu.sync_copy(z_h.at[pl.ds(sid * NRS, NRS)],
                        zst_s.at[pl.ds(sid * NRS, NRS)])
        plsc.subcore_barrier()

        pltpu.async_copy(zst_s.at[src_v.at[0]], gbuf0, semg0)

        def step(i, c):
            j0 = 2 * i
            j1 = 2 * i + 1
            pltpu.make_async_copy(zst_s.at[src_v.at[j0]], gbuf0, semg0).wait()
            scale(gbuf0, j0)

            @pl.when(i > 0)
            def _():
                pltpu.make_async_copy(gbuf1, hacc_s.at[dst_v.at[j1]],
                                      sems1).wait()
            pltpu.async_copy(zst_s.at[src_v.at[j1]], gbuf1, semg1)
            pltpu.async_copy(gbuf0, hacc_s.at[dst_v.at[j0]], sems0, add=True)
            pltpu.make_async_copy(zst_s.at[src_v.at[j1]], gbuf1, semg1).wait()
            scale(gbuf1, j1)
            pltpu.make_async_copy(gbuf0, hacc_s.at[dst_v.at[j0]],
                                  sems0).wait()

            @pl.when(i < NSTEP - 1)
            def _():
                pltpu.async_copy(zst_s.at[src_v.at[j0 + 2]], gbuf0, semg0)
            pltpu.async_copy(gbuf1, hacc_s.at[dst_v.at[j1]], sems1, add=True)
            return c
        lax.fori_loop(0, NSTEP, step, 0)
        pltpu.make_async_copy(gbuf1, hacc_s.at[dst_v.at[CH - 1]],
                              sems1).wait()
        plsc.subcore_barrier()
        pltpu.sync_copy(hacc_s.at[pl.ds(sid * RPT, RPT)],
                        out_h.at[pl.ds(sid * RPT, RPT)])

    @pl.when(cid == 0)
    def _():
        phase2(z0_h, o0_h)
        phase2(z1_h, o1_h)
        phase2(z2_h, o2_h)
        phase2(z3_h, o3_h)

    @pl.when(cid == 1)
    def _():
        phase2(z4_h, o4_h)
        phase2(z5_h, o5_h)
        phase2(z6_h, o6_h)
        phase2(z7_h, o7_h)


def kernel(x, edge_index, W, A):
    Wt = W.T
    a1 = A[0, :DOUT].reshape(DOUT, 1)
    a2 = A[0, DOUT:].reshape(DOUT, 1)
    outs = pl.pallas_call(
        _tc_body,
        grid=(N // BN,),
        in_specs=[pl.BlockSpec((BN, DIN), lambda i: (i, 0)),
                  pl.BlockSpec((DIN, DOUT), lambda i: (0, 0)),
                  pl.BlockSpec((DOUT, 1), lambda i: (0, 0)),
                  pl.BlockSpec((DOUT, 1), lambda i: (0, 0))],
        out_specs=[pl.BlockSpec((BN, Q), lambda i: (i, 0))] * NZ +
                  [pl.BlockSpec((BN, 1), lambda i: (i, 0))] * 2,
        out_shape=[jax.ShapeDtypeStruct((N, Q), jnp.float32)] * NZ +
                  [jax.ShapeDtypeStruct((N, 1), jnp.float32)] * 2,
    )(x, Wt, a1, a2)
    zs = outs[:NZ]
    s2, d2 = outs[NZ], outs[NZ + 1]

    s = jnp.pad(s2[:, 0], (0, NP - N))
    d = jnp.pad(d2[:, 0], (0, NP - N))
    src = jnp.pad(edge_index[0].reshape(NT, EPT),
                  ((0, 0), (0, EPTP - EPT))).reshape(NT, CH, G)
    dst = jnp.pad(edge_index[1].reshape(NT, EPT),
                  ((0, 0), (0, EPTP - EPT))).reshape(NT, CH, G)

    sc = pl.kernel(
        _sc_body,
        out_type=[jax.ShapeDtypeStruct((NP, Q), jnp.float32)] * NZ,
        mesh=plsc.VectorSubcoreMesh(core_axis_name="c", subcore_axis_name="s"),
        compiler_params=pltpu.CompilerParams(needs_layout_passes=False,
                                             use_tc_tiling_on_sc=False),
        scratch_types=[
            pltpu.VMEM((NP,), jnp.float32),
            pltpu.VMEM((NP,), jnp.float32),
            pltpu.VMEM((CH, G), jnp.int32),
            pltpu.VMEM((CH, G), jnp.int32),
            pltpu.VMEM((CH, G), jnp.float32),
            pltpu.VMEM((RPT,), jnp.float32),
            pltpu.VMEM((G, Q), jnp.float32),
            pltpu.VMEM((G, Q), jnp.float32),
            pltpu.VMEM_SHARED((N, Q), jnp.float32),
            pltpu.VMEM_SHARED((NP, Q), jnp.float32),
            pltpu.VMEM_SHARED((NP,), jnp.float32),
            pltpu.SemaphoreType.DMA,
            pltpu.SemaphoreType.DMA,
            pltpu.SemaphoreType.DMA,
            pltpu.SemaphoreType.DMA,
            pltpu.SemaphoreType.DMA,
        ],
    )
    o = sc(*zs, s, d, src, dst)
    return jnp.concatenate([oq[:N] for oq in o], axis=1)


# speedup vs baseline: 2.1923x; 1.1461x over previous
import jax
import jax.numpy as jnp
from jax import lax
from jax.experimental import pallas as pl
from jax.experimental.pallas import tpu as pltpu
from jax.experimental.pallas import tpu_sc as plsc

N = 10000
E = 160000
DIN = 256
DOUT = 256
Q = 32
NZ = DOUT // Q
NT = 16
L = 16
EPT = E // NT
G = 128
CH = 80
EPTP = CH * G
NP = 10240
RPT = NP // NT
NRS = N // NT
NB = RPT // G
NSTEP = CH // 2
BN = 1000


def _tc_body(x_ref, wt_ref, a1_ref, a2_ref, *out_refs):
    z = jnp.dot(x_ref[...], wt_ref[...], preferred_element_type=jnp.float32)
    for q in range(NZ):
        out_refs[q][...] = z[:, q * Q:(q + 1) * Q]
    out_refs[NZ][...] = jnp.dot(z, a1_ref[...],
                                preferred_element_type=jnp.float32)
    out_refs[NZ + 1][...] = jnp.dot(z, a2_ref[...],
                                    preferred_element_type=jnp.float32)


def _sc_body(*refs):
    (z0_h, z1_h, z2_h, z3_h, z4_h, z5_h, z6_h, z7_h, s_h, d_h, src_h, dst_h,
     out_h,
     s_v, d_v, src_v, dst_v, ee_v, invsl_v, gbuf0, gbuf1,
     zst_s, hacc_s, den_s, semd, semg0, semg1, sems0, sems1) = refs
    cid = lax.axis_index("c")
    sid = lax.axis_index("s")

    pltpu.sync_copy(s_h, s_v)
    pltpu.sync_copy(d_h, d_v)
    pltpu.sync_copy(src_h.at[sid], src_v)
    pltpu.sync_copy(dst_h.at[sid], dst_v)

    zero16 = jnp.zeros((L,), jnp.float32)
    iota16 = lax.iota(jnp.int32, L)

    def zero_invsl(i, c):
        invsl_v[pl.ds(i * L, L)] = zero16
        return c
    lax.fori_loop(0, RPT // L, zero_invsl, 0)
    pltpu.sync_copy(invsl_v, den_s.at[pl.ds(sid * RPT, RPT)])

    neg = jnp.full((L,), -1e30, jnp.float32)

    def mxs(i, acc):
        return jnp.maximum(acc, s_v[pl.ds(i * L, L)])

    def mxd(i, acc):
        return jnp.maximum(acc, d_v[pl.ds(i * L, L)])

    def lane_max(v):
        m = v[0]
        for i in range(1, L):
            m = jnp.maximum(m, v[i])
        return m
    M = lane_max(lax.fori_loop(0, NP // L, mxs, neg)) + \
        lane_max(lax.fori_loop(0, NP // L, mxd, neg))

    plsc.subcore_barrier()

    def ph1(j, c):
        for k in range(G // L):
            sl = pl.ds(k * L, L)
            s16 = src_v[j, sl]
            d16 = dst_v[j, sl]
            t = plsc.load_gather(s_v, [s16]) + plsc.load_gather(d_v, [d16])
            e = jnp.where(t >= 0, t, 0.01 * t)
            ee = jnp.exp(e - M)
            lidx = j * G + k * L + iota16
            ee = jnp.where(lidx < EPT, ee, 0.0)
            ee_v[j, sl] = ee

        @pl.when(j > 0)
        def _():
            pltpu.make_async_copy(ee_v.at[j], den_s.at[dst_v.at[j]],
                                  semd).wait()
        pltpu.async_copy(ee_v.at[j], den_s.at[dst_v.at[j]], semd, add=True)
        return c
    lax.fori_loop(0, CH, ph1, 0)
    pltpu.make_async_copy(ee_v.at[0], den_s.at[dst_v.at[0]], semd).wait()
    plsc.subcore_barrier()

    pltpu.sync_copy(den_s.at[pl.ds(sid * RPT, RPT)], invsl_v)

    def inv_loop(v, c):
        sl = pl.ds(v * L, L)
        acc = invsl_v[sl]
        invsl_v[sl] = jnp.where(acc > 0, 1.0 / acc, 1.0)
        return c
    lax.fori_loop(0, RPT // L, inv_loop, 0)
    pltpu.sync_copy(invsl_v, den_s.at[pl.ds(sid * RPT, RPT)])
    plsc.subcore_barrier()
    pltpu.sync_copy(den_s, s_v)

    @plsc.parallel_loop(0, CH, unroll=2)
    def _(j):
        for k in range(G // L):
            sl = pl.ds(k * L, L)
            iv = plsc.load_gather(s_v, [dst_v[j, sl]])
            ee_v[j, sl] = ee_v[j, sl] * iv

    gdn = lax.GatherDimensionNumbers(offset_dims=(), collapsed_slice_dims=(0,),
                                     start_index_map=(0,))

    def scale(buf, j):
        @plsc.parallel_loop(0, G // L, unroll=2)
        def _(g):
            a16 = ee_v[j, pl.ds(g * L, L)]
            for r in range(L):
                ab = lax.gather(a16, jnp.full((L, 1), r, jnp.int32), gdn,
                                (1,),
                                mode=lax.GatherScatterMode.PROMISE_IN_BOUNDS)
                row = g * L + r
                for v in range(Q // L):
                    sl2 = pl.ds(v * L, L)
                    buf[row, sl2] = buf[row, sl2] * ab

    def phase2(z_h, qoff):
        def zg(r, c):
            for v in range(Q // L):
                gbuf0[r, pl.ds(v * L, L)] = zero16
            return c
        lax.fori_loop(0, G, zg, 0)
        for b in range(NB):
            pltpu.sync_copy(gbuf0, hacc_s.at[pl.ds(sid * RPT + b * G, G)])
        pltpu.sync_copy(z_h.at[pl.ds(sid * NRS, NRS)],
                        zst_s.at[pl.ds(sid * NRS, NRS)])
        plsc.subcore_barrier()

        pltpu.async_copy(zst_s.at[src_v.at[0]], gbuf0, semg0)

        def step(i, c):
            j0 = 2 * i
            j1 = 2 * i + 1
            pltpu.make_async_copy(zst_s.at[src_v.at[j0]], gbuf0, semg0).wait()
            scale(gbuf0, j0)

            @pl.when(i > 0)
            def _():
                pltpu.make_async_copy(gbuf1, hacc_s.at[dst_v.at[j1]],
                                      sems1).wait()
            pltpu.async_copy(zst_s.at[src_v.at[j1]], gbuf1, semg1)
            pltpu.async_copy(gbuf0, hacc_s.at[dst_v.at[j0]], sems0, add=True)
            pltpu.make_async_copy(zst_s.at[src_v.at[j1]], gbuf1, semg1).wait()
            scale(gbuf1, j1)
            pltpu.make_async_copy(gbuf0, hacc_s.at[dst_v.at[j0]],
                                  sems0).wait()

            @pl.when(i < NSTEP - 1)
            def _():
                pltpu.async_copy(zst_s.at[src_v.at[j0 + 2]], gbuf0, semg0)
            pltpu.async_copy(gbuf1, hacc_s.at[dst_v.at[j1]], sems1, add=True)
            return c
        lax.fori_loop(0, NSTEP, step, 0)
        pltpu.make_async_copy(gbuf1, hacc_s.at[dst_v.at[CH - 1]],
                              sems1).wait()
        plsc.subcore_barrier()
        pltpu.sync_copy(hacc_s.at[pl.ds(sid * RPT, RPT)],
                        out_h.at[pl.ds(sid * RPT, RPT), pl.ds(qoff, Q)])

    @pl.when(cid == 0)
    def _():
        phase2(z0_h, 0 * Q)
        phase2(z1_h, 1 * Q)
        phase2(z2_h, 2 * Q)
        phase2(z3_h, 3 * Q)

    @pl.when(cid == 1)
    def _():
        phase2(z4_h, 4 * Q)
        phase2(z5_h, 5 * Q)
        phase2(z6_h, 6 * Q)
        phase2(z7_h, 7 * Q)


def kernel(x, edge_index, W, A):
    Wt = W.T
    a1 = A[0, :DOUT].reshape(DOUT, 1)
    a2 = A[0, DOUT:].reshape(DOUT, 1)
    outs = pl.pallas_call(
        _tc_body,
        grid=(N // BN,),
        in_specs=[pl.BlockSpec((BN, DIN), lambda i: (i, 0)),
                  pl.BlockSpec((DIN, DOUT), lambda i: (0, 0)),
                  pl.BlockSpec((DOUT, 1), lambda i: (0, 0)),
                  pl.BlockSpec((DOUT, 1), lambda i: (0, 0))],
        out_specs=[pl.BlockSpec((BN, Q), lambda i: (i, 0))] * NZ +
                  [pl.BlockSpec((BN, 1), lambda i: (i, 0))] * 2,
        out_shape=[jax.ShapeDtypeStruct((N, Q), jnp.float32)] * NZ +
                  [jax.ShapeDtypeStruct((N, 1), jnp.float32)] * 2,
    )(x, Wt, a1, a2)
    zs = outs[:NZ]
    s2, d2 = outs[NZ], outs[NZ + 1]

    s = jnp.pad(s2[:, 0], (0, NP - N))
    d = jnp.pad(d2[:, 0], (0, NP - N))
    src = jnp.pad(edge_index[0].reshape(NT, EPT),
                  ((0, 0), (0, EPTP - EPT))).reshape(NT, CH, G)
    dst = jnp.pad(edge_index[1].reshape(NT, EPT),
                  ((0, 0), (0, EPTP - EPT))).reshape(NT, CH, G)

    sc = pl.kernel(
        _sc_body,
        out_type=jax.ShapeDtypeStruct((NP, DOUT), jnp.float32),
        mesh=plsc.VectorSubcoreMesh(core_axis_name="c", subcore_axis_name="s"),
        compiler_params=pltpu.CompilerParams(needs_layout_passes=False,
                                             use_tc_tiling_on_sc=False),
        scratch_types=[
            pltpu.VMEM((NP,), jnp.float32),
            pltpu.VMEM((NP,), jnp.float32),
            pltpu.VMEM((CH, G), jnp.int32),
            pltpu.VMEM((CH, G), jnp.int32),
            pltpu.VMEM((CH, G), jnp.float32),
            pltpu.VMEM((RPT,), jnp.float32),
            pltpu.VMEM((G, Q), jnp.float32),
            pltpu.VMEM((G, Q), jnp.float32),
            pltpu.VMEM_SHARED((N, Q), jnp.float32),
            pltpu.VMEM_SHARED((NP, Q), jnp.float32),
            pltpu.VMEM_SHARED((NP,), jnp.float32),
            pltpu.SemaphoreType.DMA,
            pltpu.SemaphoreType.DMA,
            pltpu.SemaphoreType.DMA,
            pltpu.SemaphoreType.DMA,
            pltpu.SemaphoreType.DMA,
        ],
    )
    o = sc(*zs, s, d, src, dst)
    return o[:N]


# speedup vs baseline: 2.2079x; 1.0071x over previous
import jax
import jax.numpy as jnp
from jax import lax
from jax.experimental import pallas as pl
from jax.experimental.pallas import tpu as pltpu
from jax.experimental.pallas import tpu_sc as plsc

N = 10000
E = 160000
DIN = 256
DOUT = 256
Q = 32
NZ = DOUT // Q
NT = 16
L = 16
EPT = E // NT
G = 128
CH = 80
EPTP = CH * G
NP = 10240
RPT = NP // NT
NRS = N // NT
NB = RPT // G
NSTEP = CH // 2
BN = 1000


def _tc_body(x_ref, wt_ref, a1_ref, a2_ref, *out_refs):
    z = jnp.dot(x_ref[...], wt_ref[...], preferred_element_type=jnp.float32)
    for q in range(NZ):
        out_refs[q][...] = z[:, q * Q:(q + 1) * Q]
    out_refs[NZ][...] = jnp.dot(z, a1_ref[...],
                                preferred_element_type=jnp.float32)
    out_refs[NZ + 1][...] = jnp.dot(z, a2_ref[...],
                                    preferred_element_type=jnp.float32)


def _sc_body(*refs):
    (z0_h, z1_h, z2_h, z3_h, z4_h, z5_h, z6_h, z7_h, s_h, d_h, src_h, dst_h,
     out_h,
     s_v, d_v, src_v, dst_v, ee_v, invsl_v, gbuf0, gbuf1,
     zst_s, hacc_s, den_s, semd, semg0, semg1, sems0, sems1) = refs
    cid = lax.axis_index("c")
    sid = lax.axis_index("s")

    pltpu.sync_copy(s_h, s_v)
    pltpu.sync_copy(d_h, d_v)
    pltpu.sync_copy(src_h.at[sid], src_v)
    pltpu.sync_copy(dst_h.at[sid], dst_v)

    zero16 = jnp.zeros((L,), jnp.float32)
    iota16 = lax.iota(jnp.int32, L)

    def zero_invsl(i, c):
        invsl_v[pl.ds(i * L, L)] = zero16
        return c
    lax.fori_loop(0, RPT // L, zero_invsl, 0)
    pltpu.sync_copy(invsl_v, den_s.at[pl.ds(sid * RPT, RPT)])

    neg = jnp.full((L,), -1e30, jnp.float32)

    def mxs(i, acc):
        return jnp.maximum(acc, s_v[pl.ds(i * L, L)])

    def mxd(i, acc):
        return jnp.maximum(acc, d_v[pl.ds(i * L, L)])

    def lane_max(v):
        m = v[0]
        for i in range(1, L):
            m = jnp.maximum(m, v[i])
        return m
    M = lane_max(lax.fori_loop(0, NP // L, mxs, neg)) + \
        lane_max(lax.fori_loop(0, NP // L, mxd, neg))

    plsc.subcore_barrier()

    @plsc.parallel_loop(0, CH, unroll=2)
    def _(j):
        for k in range(G // L):
            sl = pl.ds(k * L, L)
            s16 = src_v[j, sl]
            d16 = dst_v[j, sl]
            t = plsc.load_gather(s_v, [s16]) + plsc.load_gather(d_v, [d16])
            e = jnp.where(t >= 0, t, 0.01 * t)
            ee = jnp.exp(e - M)
            lidx = j * G + k * L + iota16
            ee = jnp.where(lidx < EPT, ee, 0.0)
            ee_v[j, sl] = ee

    def ph1(j, c):
        @pl.when(j > 0)
        def _():
            pltpu.make_async_copy(ee_v.at[j], den_s.at[dst_v.at[j]],
                                  semd).wait()
        pltpu.async_copy(ee_v.at[j], den_s.at[dst_v.at[j]], semd, add=True)
        return c
    lax.fori_loop(0, CH, ph1, 0)
    pltpu.make_async_copy(ee_v.at[0], den_s.at[dst_v.at[0]], semd).wait()
    plsc.subcore_barrier()

    pltpu.sync_copy(den_s.at[pl.ds(sid * RPT, RPT)], invsl_v)

    def inv_loop(v, c):
        sl = pl.ds(v * L, L)
        acc = invsl_v[sl]
        invsl_v[sl] = jnp.where(acc > 0, 1.0 / acc, 1.0)
        return c
    lax.fori_loop(0, RPT // L, inv_loop, 0)
    pltpu.sync_copy(invsl_v, den_s.at[pl.ds(sid * RPT, RPT)])
    plsc.subcore_barrier()
    pltpu.sync_copy(den_s, s_v)

    @plsc.parallel_loop(0, CH, unroll=2)
    def _(j):
        for k in range(G // L):
            sl = pl.ds(k * L, L)
            iv = plsc.load_gather(s_v, [dst_v[j, sl]])
            ee_v[j, sl] = ee_v[j, sl] * iv

    gdn = lax.GatherDimensionNumbers(offset_dims=(), collapsed_slice_dims=(0,),
                                     start_index_map=(0,))

    def scale(buf, j):
        @plsc.parallel_loop(0, G // L, unroll=4)
        def _(g):
            a16 = ee_v[j, pl.ds(g * L, L)]
            for r in range(L):
                ab = lax.gather(a16, jnp.full((L, 1), r, jnp.int32), gdn,
                                (1,),
                                mode=lax.GatherScatterMode.PROMISE_IN_BOUNDS)
                row = g * L + r
                for v in range(Q // L):
                    sl2 = pl.ds(v * L, L)
                    buf[row, sl2] = buf[row, sl2] * ab

    def phase2(z_h, qoff):
        def zg(r, c):
            for v in range(Q // L):
                gbuf0[r, pl.ds(v * L, L)] = zero16
            return c
        lax.fori_loop(0, G, zg, 0)
        for b in range(NB):
            pltpu.sync_copy(gbuf0, hacc_s.at[pl.ds(sid * RPT + b * G, G)])
        pltpu.sync_copy(z_h.at[pl.ds(sid * NRS, NRS)],
                        zst_s.at[pl.ds(sid * NRS, NRS)])
        plsc.subcore_barrier()

        pltpu.async_copy(zst_s.at[src_v.at[0]], gbuf0, semg0)

        def step(i, c):
            j0 = 2 * i
            j1 = 2 * i + 1
            pltpu.make_async_copy(zst_s.at[src_v.at[j0]], gbuf0, semg0).wait()
            scale(gbuf0, j0)

            @pl.when(i > 0)
            def _():
                pltpu.make_async_copy(gbuf1, hacc_s.at[dst_v.at[j1]],
                                      sems1).wait()
            pltpu.async_copy(zst_s.at[src_v.at[j1]], gbuf1, semg1)
            pltpu.async_copy(gbuf0, hacc_s.at[dst_v.at[j0]], sems0, add=True)
            pltpu.make_async_copy(zst_s.at[src_v.at[j1]], gbuf1, semg1).wait()
            scale(gbuf1, j1)
            pltpu.make_async_copy(gbuf0, hacc_s.at[dst_v.at[j0]],
                                  sems0).wait()

            @pl.when(i < NSTEP - 1)
            def _():
                pltpu.async_copy(zst_s.at[src_v.at[j0 + 2]], gbuf0, semg0)
            pltpu.async_copy(gbuf1, hacc_s.at[dst_v.at[j1]], sems1, add=True)
            return c
        lax.fori_loop(0, NSTEP, step, 0)
        pltpu.make_async_copy(gbuf1, hacc_s.at[dst_v.at[CH - 1]],
                              sems1).wait()
        plsc.subcore_barrier()
        pltpu.sync_copy(hacc_s.at[pl.ds(sid * RPT, RPT)],
                        out_h.at[pl.ds(sid * RPT, RPT), pl.ds(qoff, Q)])

    @pl.when(cid == 0)
    def _():
        phase2(z0_h, 0 * Q)
        phase2(z1_h, 1 * Q)
        phase2(z2_h, 2 * Q)
        phase2(z3_h, 3 * Q)

    @pl.when(cid == 1)
    def _():
        phase2(z4_h, 4 * Q)
        phase2(z5_h, 5 * Q)
        phase2(z6_h, 6 * Q)
        phase2(z7_h, 7 * Q)


def kernel(x, edge_index, W, A):
    Wt = W.T
    a1 = A[0, :DOUT].reshape(DOUT, 1)
    a2 = A[0, DOUT:].reshape(DOUT, 1)
    outs = pl.pallas_call(
        _tc_body,
        grid=(N // BN,),
        in_specs=[pl.BlockSpec((BN, DIN), lambda i: (i, 0)),
                  pl.BlockSpec((DIN, DOUT), lambda i: (0, 0)),
                  pl.BlockSpec((DOUT, 1), lambda i: (0, 0)),
                  pl.BlockSpec((DOUT, 1), lambda i: (0, 0))],
        out_specs=[pl.BlockSpec((BN, Q), lambda i: (i, 0))] * NZ +
                  [pl.BlockSpec((BN, 1), lambda i: (i, 0))] * 2,
        out_shape=[jax.ShapeDtypeStruct((N, Q), jnp.float32)] * NZ +
                  [jax.ShapeDtypeStruct((N, 1), jnp.float32)] * 2,
    )(x, Wt, a1, a2)
    zs = outs[:NZ]
    s2, d2 = outs[NZ], outs[NZ + 1]

    s = jnp.pad(s2[:, 0], (0, NP - N))
    d = jnp.pad(d2[:, 0], (0, NP - N))
    src = jnp.pad(edge_index[0].reshape(NT, EPT),
                  ((0, 0), (0, EPTP - EPT))).reshape(NT, CH, G)
    dst = jnp.pad(edge_index[1].reshape(NT, EPT),
                  ((0, 0), (0, EPTP - EPT))).reshape(NT, CH, G)

    sc = pl.kernel(
        _sc_body,
        out_type=jax.ShapeDtypeStruct((NP, DOUT), jnp.float32),
        mesh=plsc.VectorSubcoreMesh(core_axis_name="c", subcore_axis_name="s"),
        compiler_params=pltpu.CompilerParams(needs_layout_passes=False,
                                             use_tc_tiling_on_sc=False),
        scratch_types=[
            pltpu.VMEM((NP,), jnp.float32),
            pltpu.VMEM((NP,), jnp.float32),
            pltpu.VMEM((CH, G), jnp.int32),
            pltpu.VMEM((CH, G), jnp.int32),
            pltpu.VMEM((CH, G), jnp.float32),
            pltpu.VMEM((RPT,), jnp.float32),
            pltpu.VMEM((G, Q), jnp.float32),
            pltpu.VMEM((G, Q), jnp.float32),
            pltpu.VMEM_SHARED((N, Q), jnp.float32),
            pltpu.VMEM_SHARED((NP, Q), jnp.float32),
            pltpu.VMEM_SHARED((NP,), jnp.float32),
            pltpu.SemaphoreType.DMA,
            pltpu.SemaphoreType.DMA,
            pltpu.SemaphoreType.DMA,
            pltpu.SemaphoreType.DMA,
            pltpu.SemaphoreType.DMA,
        ],
    )
    o = sc(*zs, s, d, src, dst)
    return o[:N]
